# baseline (speedup 1.0000x reference)
import dataclasses

import jax
import jax.numpy as jnp
from jax import lax
from jax.experimental import pallas as pl
from jax.experimental.pallas import tpu as pltpu
from jax.experimental.pallas import tpu_sc as plsc

_N = 100000
_E = 6400000
_NC = 2
_NS = 16
_NW = _NC * _NS
_L = 16
_B = 512
_K = 2
_G = _B * _K
_NGROUPS = _E // _G
_GBASE = _NGROUPS // _NW
_GREM = _NGROUPS - _GBASE * _NW

_NP = 100096
_NODE_CHUNK = 6272
_NODE_LAST = _NP - (_NS - 1) * _NODE_CHUNK


def _dist16(s):
    s = jnp.maximum(s, 1e-35)
    bi = plsc.bitcast(s, jnp.int32)
    r = plsc.bitcast(jnp.int32(0x5F3759DF) - (bi >> 1), jnp.float32)
    sh = 0.5 * s
    r = r * (1.5 - sh * r * r)
    r = r * (1.5 - sh * r * r)
    return s * r


def _bond_body(xyz_hbm, adj_hbm, len_hbm, par_hbm, out_hbm,
               x_sh, y_sh, z_sh, en_sh, stage_v, buf_v, bufy_v, bufz_v,
               pairs_v, src_v, dst_v,
               len_v, par_v, xs_v, ys_v, zs_v, xd_v, yd_v, zd_v, h_v,
               sem_lin, sem_gat, sem_sc):
    core = lax.axis_index("c")
    sub = lax.axis_index("s")
    wid = core * _NS + sub

    iota = lax.iota(jnp.int32, _L)
    z16 = jnp.zeros((_L,), jnp.float32)
    c0 = jnp.zeros((_L,), jnp.int32)
    c1 = jnp.full((_L,), 1, jnp.int32)
    c2 = jnp.full((_L,), 2, jnp.int32)

    noff = sub * _NODE_CHUNK
    is_last = sub == _NS - 1
    xyz_rows = _N - (_NS - 1) * _NODE_CHUNK

    @pl.when(jnp.logical_not(is_last))
    def _():
        pltpu.sync_copy(xyz_hbm.at[pl.ds(noff, _NODE_CHUNK)], stage_v)

    @pl.when(is_last)
    def _():
        pltpu.sync_copy(xyz_hbm.at[pl.ds(noff, xyz_rows)],
                        stage_v.at[pl.ds(0, xyz_rows)])

    @pl.loop(0, _NODE_CHUNK // _L)
    def _(i):
        n16 = i * _L + iota
        buf_v[pl.ds(i * _L, _L)] = plsc.load_gather(stage_v, [n16, c0])
        bufy_v[pl.ds(i * _L, _L)] = plsc.load_gather(stage_v, [n16, c1])
        bufz_v[pl.ds(i * _L, _L)] = plsc.load_gather(stage_v, [n16, c2])

    @pl.when(jnp.logical_not(is_last))
    def _():
        pltpu.sync_copy(buf_v, x_sh.at[pl.ds(noff, _NODE_CHUNK)])
        pltpu.sync_copy(bufy_v, y_sh.at[pl.ds(noff, _NODE_CHUNK)])
        pltpu.sync_copy(bufz_v, z_sh.at[pl.ds(noff, _NODE_CHUNK)])

    @pl.when(is_last)
    def _():
        pltpu.sync_copy(buf_v.at[pl.ds(0, _NODE_LAST)],
                        x_sh.at[pl.ds(noff, _NODE_LAST)])
        pltpu.sync_copy(bufy_v.at[pl.ds(0, _NODE_LAST)],
                        y_sh.at[pl.ds(noff, _NODE_LAST)])
        pltpu.sync_copy(bufz_v.at[pl.ds(0, _NODE_LAST)],
                        z_sh.at[pl.ds(noff, _NODE_LAST)])

    @pl.loop(0, _NODE_CHUNK // _L)
    def _(i):
        buf_v[pl.ds(i * _L, _L)] = z16

    @pl.when(sub < _NS - 1)
    def _():
        pltpu.sync_copy(buf_v, en_sh.at[pl.ds(noff, _NODE_CHUNK)])

    @pl.when(sub == _NS - 1)
    def _():
        pltpu.sync_copy(buf_v.at[pl.ds(0, _NODE_LAST)],
                        en_sh.at[pl.ds(noff, _NODE_LAST)])

    plsc.subcore_barrier()

    ngroups = _GBASE + jnp.where(wid < _GREM, 1, 0).astype(jnp.int32)
    g0 = _GBASE * wid + jnp.minimum(wid, _GREM)

    gathered = (xs_v, ys_v, zs_v, xd_v, yd_v, zd_v)

    @pl.loop(0, ngroups)
    def _(g):
        e0 = (g0 + g) * _G
        d_adj = pltpu.async_copy(adj_hbm.at[pl.ds(e0, _G)], pairs_v,
                                 sem_lin)
        d_len = pltpu.async_copy(len_hbm.at[pl.ds(e0, _G)], len_v, sem_lin)
        d_par = pltpu.async_copy(par_hbm.at[pl.ds(e0, _G)], par_v, sem_lin)

        def deint(j):
            srow = src_v.at[j]
            drow = dst_v.at[j]
            for jj in range(_B // _L):
                r16 = _B * j + _L * jj + iota
                ev = plsc.load_gather(pairs_v, [r16, c0])
                ov = plsc.load_gather(pairs_v, [r16, c1])
                srow[pl.ds(jj * _L, _L)] = ev
                drow[pl.ds(jj * _L, _L)] = ov

        def fire_gathers(j):
            p = j % 2
            srow = src_v.at[j]
            drow = dst_v.at[j]
            return [
                pltpu.async_copy(x_sh.at[srow], xs_v.at[p], sem_gat),
                pltpu.async_copy(y_sh.at[srow], ys_v.at[p], sem_gat),
                pltpu.async_copy(z_sh.at[srow], zs_v.at[p], sem_gat),
                pltpu.async_copy(x_sh.at[drow], xd_v.at[p], sem_gat),
                pltpu.async_copy(y_sh.at[drow], yd_v.at[p], sem_gat),
                pltpu.async_copy(z_sh.at[drow], zd_v.at[p], sem_gat),
            ]

        d_adj.wait()
        deint(0)
        pending = {0: fire_gathers(0)}
        d_len.wait()
        d_par.wait()
        scat = []
        for j in range(_K):
            if j + 1 < _K:
                deint(j + 1)
                pending[j + 1] = fire_gathers(j + 1)
            for d in pending.pop(j):
                d.wait()
            p = j % 2
            hrow = h_v.at[j]
            for jj in range(_B // _L):
                sl = pl.ds(jj * _L, _L)
                dx = xs_v.at[p][sl] - xd_v.at[p][sl]
                dy = ys_v.at[p][sl] - yd_v.at[p][sl]
                dz = zs_v.at[p][sl] - zd_v.at[p][sl]
                e = _dist16(dx * dx + dy * dy + dz * dz)
                o16 = j * _B + jj * _L + iota
                d = e - plsc.load_gather(len_v, [o16, c0])
                h = plsc.load_gather(par_v, [o16, c0]) * d
                h = h * d
                hrow[sl] = h * 0.5
            scat.append(pltpu.async_copy(hrow, en_sh.at[src_v.at[j]],
                                         sem_sc, add=True))
            scat.append(pltpu.async_copy(hrow, en_sh.at[dst_v.at[j]],
                                         sem_sc, add=True))
        for d in scat:
            d.wait()

    plsc.subcore_barrier()

    @pl.when(sub < _NS - 1)
    def _():
        pltpu.sync_copy(en_sh.at[pl.ds(noff, _NODE_CHUNK)], buf_v)
        pltpu.sync_copy(buf_v, out_hbm.at[core].at[pl.ds(noff, _NODE_CHUNK)])

    @pl.when(sub == _NS - 1)
    def _():
        pltpu.sync_copy(en_sh.at[pl.ds(noff, _NODE_LAST)],
                        buf_v.at[pl.ds(0, _NODE_LAST)])
        pltpu.sync_copy(buf_v.at[pl.ds(0, _NODE_LAST)],
                        out_hbm.at[core].at[pl.ds(noff, _NODE_LAST)])


def kernel(xyz, bond_adj, bond_len, bond_par):
    adj = bond_adj.astype(jnp.int32)
    mesh = plsc.VectorSubcoreMesh(core_axis_name="c", subcore_axis_name="s")
    cp = pltpu.CompilerParams()
    if "needs_layout_passes" in pltpu.CompilerParams.__dataclass_fields__:
        cp = dataclasses.replace(cp, needs_layout_passes=False)
    if "use_tc_tiling_on_sc" in pltpu.CompilerParams.__dataclass_fields__:
        cp = dataclasses.replace(cp, use_tc_tiling_on_sc=False)
    run = pl.kernel(
        _bond_body,
        out_type=jax.ShapeDtypeStruct((_NC, _NP), jnp.float32),
        mesh=mesh,
        compiler_params=cp,
        scratch_types=[
            pltpu.VMEM_SHARED((_NP,), jnp.float32),
            pltpu.VMEM_SHARED((_NP,), jnp.float32),
            pltpu.VMEM_SHARED((_NP,), jnp.float32),
            pltpu.VMEM_SHARED((_NP,), jnp.float32),
            pltpu.VMEM((_NODE_CHUNK, 3), jnp.float32),
            pltpu.VMEM((_NODE_CHUNK,), jnp.float32),
            pltpu.VMEM((_NODE_CHUNK,), jnp.float32),
            pltpu.VMEM((_NODE_CHUNK,), jnp.float32),
            pltpu.VMEM((_G, 2), jnp.int32),
            pltpu.VMEM((_K, _B), jnp.int32),
            pltpu.VMEM((_K, _B), jnp.int32),
            pltpu.VMEM((_G, 1), jnp.float32),
            pltpu.VMEM((_G, 1), jnp.float32),
            pltpu.VMEM((2, _B), jnp.float32),
            pltpu.VMEM((2, _B), jnp.float32),
            pltpu.VMEM((2, _B), jnp.float32),
            pltpu.VMEM((2, _B), jnp.float32),
            pltpu.VMEM((2, _B), jnp.float32),
            pltpu.VMEM((2, _B), jnp.float32),
            pltpu.VMEM((_K, _B), jnp.float32),
            pltpu.SemaphoreType.DMA,
            pltpu.SemaphoreType.DMA,
            pltpu.SemaphoreType.DMA,
        ],
    )
    partials = run(xyz, adj, bond_len, bond_par)
    return (partials[0, :_N] + partials[1, :_N]).reshape(_N, 1)


# speedup vs baseline: 2.9950x; 2.9950x over previous
import dataclasses

import jax
import jax.numpy as jnp
from jax import lax
from jax.experimental import pallas as pl
from jax.experimental.pallas import tpu as pltpu
from jax.experimental.pallas import tpu_sc as plsc

_N = 100000
_E = 6400000
_NC = 2
_NS = 16
_NW = _NC * _NS
_L = 16
_B = 512
_K = 2
_G = _B * _K
_NGROUPS = _E // _G
_GBASE = _NGROUPS // _NW
_GREM = _NGROUPS - _GBASE * _NW

_NP = 100096
_NODE_CHUNK = 6272
_NODE_LAST = _NP - (_NS - 1) * _NODE_CHUNK


def _dist16(s):
    s = jnp.maximum(s, 1e-35)
    bi = plsc.bitcast(s, jnp.int32)
    r = plsc.bitcast(jnp.int32(0x5F3759DF) - (bi >> 1), jnp.float32)
    sh = 0.5 * s
    r = r * (1.5 - sh * r * r)
    r = r * (1.5 - sh * r * r)
    return s * r


def _bond_body(xyz_hbm, adj_hbm, len_hbm, par_hbm, out_hbm,
               x_sh, y_sh, z_sh, en_sh, stage_v, buf_v, bufy_v, bufz_v,
               pairs_v, src_v, dst_v,
               len_v, par_v, xs_v, ys_v, zs_v, xd_v, yd_v, zd_v, h_v,
               sem_lin, sem_gat, sem_sc):
    core = lax.axis_index("c")
    sub = lax.axis_index("s")
    wid = core * _NS + sub

    iota = lax.iota(jnp.int32, _L)
    z16 = jnp.zeros((_L,), jnp.float32)
    c0 = jnp.zeros((_L,), jnp.int32)
    c1 = jnp.full((_L,), 1, jnp.int32)
    c2 = jnp.full((_L,), 2, jnp.int32)

    noff = sub * _NODE_CHUNK
    is_last = sub == _NS - 1
    xyz_rows = _N - (_NS - 1) * _NODE_CHUNK

    @pl.when(jnp.logical_not(is_last))
    def _():
        pltpu.sync_copy(xyz_hbm.at[pl.ds(noff, _NODE_CHUNK)], stage_v)

    @pl.when(is_last)
    def _():
        pltpu.sync_copy(xyz_hbm.at[pl.ds(noff, xyz_rows)],
                        stage_v.at[pl.ds(0, xyz_rows)])

    @pl.loop(0, _NODE_CHUNK // _L)
    def _(i):
        n16 = i * _L + iota
        buf_v[pl.ds(i * _L, _L)] = plsc.load_gather(stage_v, [n16, c0])
        bufy_v[pl.ds(i * _L, _L)] = plsc.load_gather(stage_v, [n16, c1])
        bufz_v[pl.ds(i * _L, _L)] = plsc.load_gather(stage_v, [n16, c2])

    @pl.when(jnp.logical_not(is_last))
    def _():
        pltpu.sync_copy(buf_v, x_sh.at[pl.ds(noff, _NODE_CHUNK)])
        pltpu.sync_copy(bufy_v, y_sh.at[pl.ds(noff, _NODE_CHUNK)])
        pltpu.sync_copy(bufz_v, z_sh.at[pl.ds(noff, _NODE_CHUNK)])

    @pl.when(is_last)
    def _():
        pltpu.sync_copy(buf_v.at[pl.ds(0, _NODE_LAST)],
                        x_sh.at[pl.ds(noff, _NODE_LAST)])
        pltpu.sync_copy(bufy_v.at[pl.ds(0, _NODE_LAST)],
                        y_sh.at[pl.ds(noff, _NODE_LAST)])
        pltpu.sync_copy(bufz_v.at[pl.ds(0, _NODE_LAST)],
                        z_sh.at[pl.ds(noff, _NODE_LAST)])

    @pl.loop(0, _NODE_CHUNK // _L)
    def _(i):
        buf_v[pl.ds(i * _L, _L)] = z16

    @pl.when(sub < _NS - 1)
    def _():
        pltpu.sync_copy(buf_v, en_sh.at[pl.ds(noff, _NODE_CHUNK)])

    @pl.when(sub == _NS - 1)
    def _():
        pltpu.sync_copy(buf_v.at[pl.ds(0, _NODE_LAST)],
                        en_sh.at[pl.ds(noff, _NODE_LAST)])

    plsc.subcore_barrier()

    ngroups = _GBASE + jnp.where(wid < _GREM, 1, 0).astype(jnp.int32)
    g0 = _GBASE * wid + jnp.minimum(wid, _GREM)

    gathered = (xs_v, ys_v, zs_v, xd_v, yd_v, zd_v)

    @pl.loop(0, ngroups)
    def _(g):
        e0 = (g0 + g) * _G
        d_adj = pltpu.async_copy(adj_hbm.at[pl.ds(2 * e0, 2 * _G)],
                                 pairs_v, sem_lin)
        d_len = pltpu.async_copy(len_hbm.at[pl.ds(e0, _G)], len_v, sem_lin)
        d_par = pltpu.async_copy(par_hbm.at[pl.ds(e0, _G)], par_v, sem_lin)

        def deint(j):
            srow = src_v.at[j]
            drow = dst_v.at[j]
            for jj in range(_B // _L):
                fl = 2 * _B * j + 2 * _L * jj
                ev = plsc.load_gather(pairs_v, [fl + 2 * iota])
                ov = plsc.load_gather(pairs_v, [fl + 1 + 2 * iota])
                srow[pl.ds(jj * _L, _L)] = ev
                drow[pl.ds(jj * _L, _L)] = ov

        def fire_gathers(j):
            p = j % 2
            srow = src_v.at[j]
            drow = dst_v.at[j]
            return [
                pltpu.async_copy(x_sh.at[srow], xs_v.at[p], sem_gat),
                pltpu.async_copy(y_sh.at[srow], ys_v.at[p], sem_gat),
                pltpu.async_copy(z_sh.at[srow], zs_v.at[p], sem_gat),
                pltpu.async_copy(x_sh.at[drow], xd_v.at[p], sem_gat),
                pltpu.async_copy(y_sh.at[drow], yd_v.at[p], sem_gat),
                pltpu.async_copy(z_sh.at[drow], zd_v.at[p], sem_gat),
            ]

        d_adj.wait()
        deint(0)
        pending = {0: fire_gathers(0)}
        d_len.wait()
        d_par.wait()
        scat = []
        for j in range(_K):
            if j + 1 < _K:
                deint(j + 1)
                pending[j + 1] = fire_gathers(j + 1)
            for d in pending.pop(j):
                d.wait()
            p = j % 2
            hrow = h_v.at[j]
            for jj in range(_B // _L):
                sl = pl.ds(jj * _L, _L)
                dx = xs_v.at[p][sl] - xd_v.at[p][sl]
                dy = ys_v.at[p][sl] - yd_v.at[p][sl]
                dz = zs_v.at[p][sl] - zd_v.at[p][sl]
                e = _dist16(dx * dx + dy * dy + dz * dz)
                off = j * _B + jj * _L
                d = e - len_v[pl.ds(off, _L)]
                h = par_v[pl.ds(off, _L)] * d
                h = h * d
                hrow[sl] = h * 0.5
            scat.append(pltpu.async_copy(hrow, en_sh.at[src_v.at[j]],
                                         sem_sc, add=True))
            scat.append(pltpu.async_copy(hrow, en_sh.at[dst_v.at[j]],
                                         sem_sc, add=True))
        for d in scat:
            d.wait()

    plsc.subcore_barrier()

    @pl.when(sub < _NS - 1)
    def _():
        pltpu.sync_copy(en_sh.at[pl.ds(noff, _NODE_CHUNK)], buf_v)
        pltpu.sync_copy(buf_v, out_hbm.at[core].at[pl.ds(noff, _NODE_CHUNK)])

    @pl.when(sub == _NS - 1)
    def _():
        pltpu.sync_copy(en_sh.at[pl.ds(noff, _NODE_LAST)],
                        buf_v.at[pl.ds(0, _NODE_LAST)])
        pltpu.sync_copy(buf_v.at[pl.ds(0, _NODE_LAST)],
                        out_hbm.at[core].at[pl.ds(noff, _NODE_LAST)])


def kernel(xyz, bond_adj, bond_len, bond_par):
    adj = jnp.minimum(bond_adj.astype(jnp.int32), _N - 1).reshape(-1)
    lenf = bond_len.reshape(-1)
    parf = bond_par.reshape(-1)
    mesh = plsc.VectorSubcoreMesh(core_axis_name="c", subcore_axis_name="s")
    cp = pltpu.CompilerParams()
    if "needs_layout_passes" in pltpu.CompilerParams.__dataclass_fields__:
        cp = dataclasses.replace(cp, needs_layout_passes=False)
    if "use_tc_tiling_on_sc" in pltpu.CompilerParams.__dataclass_fields__:
        cp = dataclasses.replace(cp, use_tc_tiling_on_sc=False)
    run = pl.kernel(
        _bond_body,
        out_type=jax.ShapeDtypeStruct((_NC, _NP), jnp.float32),
        mesh=mesh,
        compiler_params=cp,
        scratch_types=[
            pltpu.VMEM_SHARED((_NP,), jnp.float32),
            pltpu.VMEM_SHARED((_NP,), jnp.float32),
            pltpu.VMEM_SHARED((_NP,), jnp.float32),
            pltpu.VMEM_SHARED((_NP,), jnp.float32),
            pltpu.VMEM((_NODE_CHUNK, 3), jnp.float32),
            pltpu.VMEM((_NODE_CHUNK,), jnp.float32),
            pltpu.VMEM((_NODE_CHUNK,), jnp.float32),
            pltpu.VMEM((_NODE_CHUNK,), jnp.float32),
            pltpu.VMEM((2 * _G,), jnp.int32),
            pltpu.VMEM((_K, _B), jnp.int32),
            pltpu.VMEM((_K, _B), jnp.int32),
            pltpu.VMEM((_G,), jnp.float32),
            pltpu.VMEM((_G,), jnp.float32),
            pltpu.VMEM((2, _B), jnp.float32),
            pltpu.VMEM((2, _B), jnp.float32),
            pltpu.VMEM((2, _B), jnp.float32),
            pltpu.VMEM((2, _B), jnp.float32),
            pltpu.VMEM((2, _B), jnp.float32),
            pltpu.VMEM((2, _B), jnp.float32),
            pltpu.VMEM((_K, _B), jnp.float32),
            pltpu.SemaphoreType.DMA,
            pltpu.SemaphoreType.DMA,
            pltpu.SemaphoreType.DMA,
        ],
    )
    partials = run(xyz, adj, lenf, parf)
    return (partials[0, :_N] + partials[1, :_N]).reshape(_N, 1)


# speedup vs baseline: 26.6900x; 8.9115x over previous
import dataclasses

import jax
import jax.numpy as jnp
from jax import lax
from jax.experimental import pallas as pl
from jax.experimental.pallas import tpu as pltpu
from jax.experimental.pallas import tpu_sc as plsc

_N = 100000
_E = 6400000
_NC = 2
_NS = 16
_NW = _NC * _NS
_L = 16
_B = 128
_K = 4
_G = _B * _K
_NGROUPS = _E // _G
_GBASE = _NGROUPS // _NW
_GREM = _NGROUPS - _GBASE * _NW

_NP = 100096
_NODE_CHUNK = 6272
_NODE_LAST = _NP - (_NS - 1) * _NODE_CHUNK


def _dist16(s):
    s = jnp.maximum(s, 1e-35)
    bi = plsc.bitcast(s, jnp.int32)
    r = plsc.bitcast(jnp.int32(0x5F3759DF) - (bi >> 1), jnp.float32)
    sh = 0.5 * s
    r = r * (1.5 - sh * r * r)
    r = r * (1.5 - sh * r * r)
    return s * r


def _bond_body(xyz_hbm, adj_hbm, len_hbm, par_hbm, out_hbm,
               x_sh, y_sh, z_sh, en_sh, stage_v, buf_v, bufy_v, bufz_v,
               pairs_v,
               len_v, par_v, xs_v, ys_v, zs_v, xd_v, yd_v, zd_v, h_v,
               sem_lin, sem_gat, sem_sc):
    core = lax.axis_index("c")
    sub = lax.axis_index("s")
    wid = core * _NS + sub

    iota = lax.iota(jnp.int32, _L)
    z16 = jnp.zeros((_L,), jnp.float32)
    c0 = jnp.zeros((_L,), jnp.int32)
    c1 = jnp.full((_L,), 1, jnp.int32)
    c2 = jnp.full((_L,), 2, jnp.int32)

    noff = sub * _NODE_CHUNK
    is_last = sub == _NS - 1
    xyz_rows = _N - (_NS - 1) * _NODE_CHUNK

    @pl.when(jnp.logical_not(is_last))
    def _():
        pltpu.sync_copy(xyz_hbm.at[pl.ds(noff, _NODE_CHUNK)], stage_v)

    @pl.when(is_last)
    def _():
        pltpu.sync_copy(xyz_hbm.at[pl.ds(noff, xyz_rows)],
                        stage_v.at[pl.ds(0, xyz_rows)])

    @pl.loop(0, _NODE_CHUNK // _L)
    def _(i):
        n16 = i * _L + iota
        buf_v[pl.ds(i * _L, _L)] = plsc.load_gather(stage_v, [n16, c0])
        bufy_v[pl.ds(i * _L, _L)] = plsc.load_gather(stage_v, [n16, c1])
        bufz_v[pl.ds(i * _L, _L)] = plsc.load_gather(stage_v, [n16, c2])

    @pl.when(jnp.logical_not(is_last))
    def _():
        pltpu.sync_copy(buf_v, x_sh.at[pl.ds(noff, _NODE_CHUNK)])
        pltpu.sync_copy(bufy_v, y_sh.at[pl.ds(noff, _NODE_CHUNK)])
        pltpu.sync_copy(bufz_v, z_sh.at[pl.ds(noff, _NODE_CHUNK)])

    @pl.when(is_last)
    def _():
        pltpu.sync_copy(buf_v.at[pl.ds(0, _NODE_LAST)],
                        x_sh.at[pl.ds(noff, _NODE_LAST)])
        pltpu.sync_copy(bufy_v.at[pl.ds(0, _NODE_LAST)],
                        y_sh.at[pl.ds(noff, _NODE_LAST)])
        pltpu.sync_copy(bufz_v.at[pl.ds(0, _NODE_LAST)],
                        z_sh.at[pl.ds(noff, _NODE_LAST)])

    @pl.loop(0, _NODE_CHUNK // _L)
    def _(i):
        buf_v[pl.ds(i * _L, _L)] = z16

    @pl.when(sub < _NS - 1)
    def _():
        pltpu.sync_copy(buf_v, en_sh.at[pl.ds(noff, _NODE_CHUNK)])

    @pl.when(sub == _NS - 1)
    def _():
        pltpu.sync_copy(buf_v.at[pl.ds(0, _NODE_LAST)],
                        en_sh.at[pl.ds(noff, _NODE_LAST)])

    plsc.subcore_barrier()

    ngroups = _GBASE + jnp.where(wid < _GREM, 1, 0).astype(jnp.int32)
    g0 = _GBASE * wid + jnp.minimum(wid, _GREM)

    gathered = (xs_v, ys_v, zs_v, xd_v, yd_v, zd_v)

    @pl.loop(0, ngroups)
    def _(g):
        e0 = (g0 + g) * _G
        d_adj = pltpu.async_copy(adj_hbm.at[pl.ds(2 * _K * (g0 + g), 2 * _K)],
                                 pairs_v, sem_lin)
        d_len = pltpu.async_copy(len_hbm.at[pl.ds(e0, _G)], len_v, sem_lin)
        d_par = pltpu.async_copy(par_hbm.at[pl.ds(e0, _G)], par_v, sem_lin)

        def fire_gathers(j):
            p = j % 2
            srow = pairs_v.at[2 * j]
            drow = pairs_v.at[2 * j + 1]
            return [
                pltpu.async_copy(x_sh.at[srow], xs_v.at[p], sem_gat),
                pltpu.async_copy(y_sh.at[srow], ys_v.at[p], sem_gat),
                pltpu.async_copy(z_sh.at[srow], zs_v.at[p], sem_gat),
                pltpu.async_copy(x_sh.at[drow], xd_v.at[p], sem_gat),
                pltpu.async_copy(y_sh.at[drow], yd_v.at[p], sem_gat),
                pltpu.async_copy(z_sh.at[drow], zd_v.at[p], sem_gat),
            ]

        d_adj.wait()
        pending = {0: fire_gathers(0)}
        d_len.wait()
        d_par.wait()
        scat = []
        for j in range(_K):
            if j + 1 < _K:
                pending[j + 1] = fire_gathers(j + 1)
            for d in pending.pop(j):
                d.wait()
            p = j % 2
            hrow = h_v.at[j]
            for jj in range(_B // _L):
                sl = pl.ds(jj * _L, _L)
                dx = xs_v.at[p][sl] - xd_v.at[p][sl]
                dy = ys_v.at[p][sl] - yd_v.at[p][sl]
                dz = zs_v.at[p][sl] - zd_v.at[p][sl]
                e = _dist16(dx * dx + dy * dy + dz * dz)
                off = j * _B + jj * _L
                d = e - len_v[pl.ds(off, _L)]
                h = par_v[pl.ds(off, _L)] * d
                h = h * d
                hrow[sl] = h * 0.5
            scat.append(pltpu.async_copy(hrow, en_sh.at[pairs_v.at[2 * j]],
                                         sem_sc, add=True))
            scat.append(pltpu.async_copy(hrow, en_sh.at[pairs_v.at[2 * j + 1]],
                                         sem_sc, add=True))
        for d in scat:
            d.wait()

    plsc.subcore_barrier()

    @pl.when(sub < _NS - 1)
    def _():
        pltpu.sync_copy(en_sh.at[pl.ds(noff, _NODE_CHUNK)], buf_v)
        pltpu.sync_copy(buf_v, out_hbm.at[core].at[pl.ds(noff, _NODE_CHUNK)])

    @pl.when(sub == _NS - 1)
    def _():
        pltpu.sync_copy(en_sh.at[pl.ds(noff, _NODE_LAST)],
                        buf_v.at[pl.ds(0, _NODE_LAST)])
        pltpu.sync_copy(buf_v.at[pl.ds(0, _NODE_LAST)],
                        out_hbm.at[core].at[pl.ds(noff, _NODE_LAST)])


def kernel(xyz, bond_adj, bond_len, bond_par):
    adj = (bond_adj.astype(jnp.int32)
           .reshape(_E // _B, _B, 2).swapaxes(1, 2)
           .reshape(2 * _E // _B, _B))
    lenf = bond_len.reshape(-1)
    parf = bond_par.reshape(-1)
    mesh = plsc.VectorSubcoreMesh(core_axis_name="c", subcore_axis_name="s")
    cp = pltpu.CompilerParams()
    if "needs_layout_passes" in pltpu.CompilerParams.__dataclass_fields__:
        cp = dataclasses.replace(cp, needs_layout_passes=False)
    if "use_tc_tiling_on_sc" in pltpu.CompilerParams.__dataclass_fields__:
        cp = dataclasses.replace(cp, use_tc_tiling_on_sc=False)
    run = pl.kernel(
        _bond_body,
        out_type=jax.ShapeDtypeStruct((_NC, _NP), jnp.float32),
        mesh=mesh,
        compiler_params=cp,
        scratch_types=[
            pltpu.VMEM_SHARED((_NP,), jnp.float32),
            pltpu.VMEM_SHARED((_NP,), jnp.float32),
            pltpu.VMEM_SHARED((_NP,), jnp.float32),
            pltpu.VMEM_SHARED((_NP,), jnp.float32),
            pltpu.VMEM((_NODE_CHUNK, 3), jnp.float32),
            pltpu.VMEM((_NODE_CHUNK,), jnp.float32),
            pltpu.VMEM((_NODE_CHUNK,), jnp.float32),
            pltpu.VMEM((_NODE_CHUNK,), jnp.float32),
            pltpu.VMEM((2 * _K, _B), jnp.int32),
            pltpu.VMEM((_G,), jnp.float32),
            pltpu.VMEM((_G,), jnp.float32),
            pltpu.VMEM((2, _B), jnp.float32),
            pltpu.VMEM((2, _B), jnp.float32),
            pltpu.VMEM((2, _B), jnp.float32),
            pltpu.VMEM((2, _B), jnp.float32),
            pltpu.VMEM((2, _B), jnp.float32),
            pltpu.VMEM((2, _B), jnp.float32),
            pltpu.VMEM((_K, _B), jnp.float32),
            pltpu.SemaphoreType.DMA,
            pltpu.SemaphoreType.DMA,
            pltpu.SemaphoreType.DMA,
        ],
    )
    partials = run(xyz, adj, lenf, parf)
    return (partials[0, :_N] + partials[1, :_N]).reshape(_N, 1)
